# baseline (speedup 1.0000x reference)
import jax
import jax.numpy as jnp
from jax import lax
from jax.experimental import pallas as pl
from jax.experimental.pallas import tpu as pltpu
from jax.experimental.pallas import tpu_sc as plsc

H, W = 512, 512
HW = H * W
C = 64
P = 20000

NC, NS = 2, 16
NW = NC * NS
TILE_RANGE = HW // NW
BS = 512
NB = TILE_RANGE // BS
CHUNK = 2000
NCHUNK = P // CHUNK
LANES = 16
GR = 64


def _scatter_body(coords_hbm, feat_hbm, out_hbm,
                  cbuf, owner, subA, subB, idxbuf, stage, blockA, blockB,
                  gsem, semA, semB):
    wid = lax.axis_index("s") * NC + lax.axis_index("c")
    r0 = wid * TILE_RANGE
    iota = lax.iota(jnp.int32, LANES)
    zeros16 = jnp.zeros((LANES,), jnp.float32)
    neg16 = jnp.full((LANES,), -1, jnp.int32)
    zero_i = jnp.zeros((LANES,), jnp.int32)
    one_i = jnp.full((LANES,), 1, jnp.int32)

    def init_owner(k, _):
        owner[pl.ds(k * LANES, LANES)] = neg16
        return 0
    lax.fori_loop(0, TILE_RANGE // LANES, init_owner, 0)

    for buf in (blockA, blockB):
        def zb(c, _, buf=buf):
            def zi(k, _):
                buf[c, pl.ds(k * LANES, LANES)] = zeros16
                return 0
            return lax.fori_loop(0, BS // LANES, zi, 0)
        lax.fori_loop(0, C, zb, 0)

    def chunk_body(ck, _):
        pltpu.sync_copy(coords_hbm.at[pl.ds(ck * CHUNK, CHUNK), :], cbuf)

        def vec_body(v, _):
            pt = v * LANES + iota
            x = plsc.load_gather(cbuf, [pt, zero_i])
            y = plsc.load_gather(cbuf, [pt, one_i])
            off = (y * W + x) - r0
            m = (off >= 0) & (off < TILE_RANGE)
            offc = jnp.where(m, off, 0)
            i_vec = ck * CHUNK + pt
            plsc.store_scatter(owner, [offc], i_vec, mask=m)
            w = plsc.load_gather(owner, [offc], mask=m)
            bad = m & (w != i_vec)
            nbad = plsc.all_reduce_population_count(bad)[0]

            @pl.when(nbad > 0)
            def _fixup():
                for p in range(LANES):
                    plsc.store_scatter(owner, [offc], i_vec,
                                       mask=m & (iota == p))
            return 0

        lax.fori_loop(0, CHUNK // LANES, vec_body, 0)
        return 0

    lax.fori_loop(0, NCHUNK, chunk_body, 0)

    descs = [None] * NB
    msave = [None] * NB

    for b in range(NB):
        buf, sub, sem = (blockA, subA, semA) if b % 2 == 0 else \
                        (blockB, subB, semB)

        if b >= 2:
            descs[b - 2].wait()
            mprev = msave[b - 2]

            def zgrp(g, _, sub=sub, buf=buf, mprev=mprev):
                eg = sub[pl.ds(g * LANES, LANES)]
                gv = (g * LANES + iota) < mprev
                jv = jnp.where(gv, eg & (BS - 1), 0)

                def zch(c, _):
                    cvec = jnp.full((LANES,), c, jnp.int32)
                    plsc.store_scatter(buf, [cvec, jv], zeros16, mask=gv)
                    return 0

                lax.fori_loop(0, C, zch, 0)
                return 0

            lax.fori_loop(0, (mprev + LANES - 1) // LANES, zgrp, 0)

        def scan_body(v, mm, sub=sub, b=b):
            w = owner[pl.ds(b * BS + v * LANES, LANES)]
            sel = w >= 0
            e = w * BS + (v * LANES + iota)
            pos = mm + plsc.cumsum(sel.astype(jnp.int32)) - 1
            plsc.store_scatter(sub, [pos], e, mask=sel)
            return mm + plsc.all_reduce_population_count(sel)[0]

        m_b = lax.fori_loop(0, BS // LANES, scan_body, jnp.int32(0))
        msave[b] = m_b

        def chunk_gather(cg, _, sub=sub, buf=buf, m_b=m_b):
            base = cg * GR
            for t in range(GR // LANES):
                ev = sub[pl.ds(base + t * LANES, LANES)]
                gv = (base + t * LANES + iota) < m_b
                idxbuf[pl.ds(t * LANES, LANES)] = jnp.where(gv, ev >> 9, 0)
            pltpu.async_copy(feat_hbm.at[idxbuf], stage, gsem).wait()
            for t in range(GR // LANES):
                ev = sub[pl.ds(base + t * LANES, LANES)]
                gv = (base + t * LANES + iota) < m_b
                jv = jnp.where(gv, ev & (BS - 1), 0)
                rvec = t * LANES + iota

                def ch_body(c, _, jv=jv, gv=gv, rvec=rvec):
                    cvec = jnp.full((LANES,), c, jnp.int32)
                    vals = plsc.load_gather(stage, [rvec, cvec])
                    plsc.store_scatter(buf, [cvec, jv], vals, mask=gv)
                    return 0

                lax.fori_loop(0, C, ch_body, 0)
            return 0

        lax.fori_loop(0, (m_b + GR - 1) // GR, chunk_gather, 0)

        descs[b] = pltpu.async_copy(
            buf, out_hbm.at[:, pl.ds(r0 + b * BS, BS)], sem)

    descs[NB - 2].wait()
    descs[NB - 1].wait()


@jax.jit
def _scatter(coords, feat):
    mesh = plsc.VectorSubcoreMesh(core_axis_name="c", subcore_axis_name="s",
                                  num_cores=NC, num_subcores=NS)
    return pl.kernel(
        _scatter_body,
        out_type=jax.ShapeDtypeStruct((C, HW), jnp.float32),
        mesh=mesh,
        compiler_params=pltpu.CompilerParams(needs_layout_passes=False,
                                             use_tc_tiling_on_sc=False),
        scratch_types=[
            pltpu.VMEM((CHUNK, 2), jnp.int32),
            pltpu.VMEM((TILE_RANGE,), jnp.int32),
            pltpu.VMEM((BS,), jnp.int32),
            pltpu.VMEM((BS,), jnp.int32),
            pltpu.VMEM((GR,), jnp.int32),
            pltpu.VMEM((GR, C), jnp.float32),
            pltpu.VMEM((C, BS), jnp.float32),
            pltpu.VMEM((C, BS), jnp.float32),
            pltpu.SemaphoreType.DMA,
            pltpu.SemaphoreType.DMA,
            pltpu.SemaphoreType.DMA,
        ],
    )(coords, feat)


def kernel(pillar_features, coords):
    canvas = _scatter(jnp.asarray(coords, jnp.int32), pillar_features)
    return canvas.reshape(1, C, H, W)


# speedup vs baseline: 1.1237x; 1.1237x over previous
import jax
import jax.numpy as jnp
from jax import lax
from jax.experimental import pallas as pl
from jax.experimental.pallas import tpu as pltpu
from jax.experimental.pallas import tpu_sc as plsc

H, W = 512, 512
HW = H * W
C = 64
P = 20000

NC, NS = 2, 16
NW = NC * NS
TILE_RANGE = HW // NW
BS = 512
NB = TILE_RANGE // BS
CHUNK = 4000
NCHUNK = P // CHUNK
LANES = 16
GC = 64


def _scatter_body(xs_hbm, ys_hbm, feat_hbm, out_hbm,
                  xb0, yb0, xb1, yb1, owner, ent,
                  idx0, idx1, stage0, stage1, buf0, buf1,
                  csx0, csy0, csx1, csy1, gs0, gs1, os0, os1):
    wid = lax.axis_index("s") * NC + lax.axis_index("c")
    r0 = wid * TILE_RANGE
    iota = lax.iota(jnp.int32, LANES)
    zeros16 = jnp.zeros((LANES,), jnp.float32)
    neg16 = jnp.full((LANES,), -1, jnp.int32)

    @plsc.parallel_loop(0, TILE_RANGE, step=LANES, unroll=8)
    def _(i):
        owner[pl.ds(i, LANES)] = neg16

    for buf in (buf0, buf1):
        @plsc.parallel_loop(0, C * BS, step=LANES, unroll=4)
        def _(i, buf=buf):
            buf[i >> 9, pl.ds(i & (BS - 1), LANES)] = zeros16

    slots = ((xb0, yb0, csx0, csy0), (xb1, yb1, csx1, csy1))

    def issue_coords(ck):
        xb, yb, csx, csy = slots[ck % 2]
        s = pl.ds(ck * CHUNK, CHUNK)
        return (pltpu.async_copy(xs_hbm.at[s], xb, csx),
                pltpu.async_copy(ys_hbm.at[s], yb, csy))

    pend = issue_coords(0)
    for ck in range(NCHUNK):
        xb, yb, _, _ = slots[ck % 2]
        pend[0].wait()
        pend[1].wait()
        if ck + 1 < NCHUNK:
            pend = issue_coords(ck + 1)

        def vec_body(v, _, xb=xb, yb=yb, base=ck * CHUNK):
            for u in range(2):
                s = v * (2 * LANES) + u * LANES
                x = xb[pl.ds(s, LANES)]
                y = yb[pl.ds(s, LANES)]
                off = (y << 9) + x - r0
                m = (off >= 0) & (off < TILE_RANGE)
                offc = jnp.where(m, off, 0)
                _, lastm = plsc.scan_count(offc, mask=m)
                plsc.store_scatter(owner, [offc], base + s + iota,
                                   mask=m & lastm)
            return 0

        lax.fori_loop(0, CHUNK // (2 * LANES), vec_body, 0)

    msave = []
    for b in range(NB):
        def scan_body(v, mm, b=b):
            w = owner[pl.ds(b * BS + v * LANES, LANES)]
            sel = w >= 0
            e = (w << 9) + (v * LANES + iota)
            plsc.store_compressed(ent.at[pl.ds(b * BS + mm, LANES)], e,
                                  mask=sel)
            return mm + plsc.all_reduce_population_count(sel)[0]

        msave.append(lax.fori_loop(0, BS // LANES, scan_body, jnp.int32(0)))

    def fill_idx(b, m_b, idxr):
        def g(gi, _, b=b):
            gl = gi * LANES + iota
            e = ent[pl.ds(b * BS + gi * LANES, LANES)]
            idxr[pl.ds(gi * LANES, LANES)] = jnp.where(gl < m_b, e >> 9, 0)
            return 0
        lax.fori_loop(0, GC // LANES, g, 0)

    def process(b, stage, buf, base, cnt):
        def grp(g, _, b=b):
            gl = g * LANES + iota
            gv = gl < cnt
            e = ent[pl.ds(b * BS + base + g * LANES, LANES)]
            jv = jnp.where(gv, e & (BS - 1), 0)
            rvec = gl

            @plsc.parallel_loop(0, C, step=1, unroll=4)
            def _(k):
                cvec = (k + iota) & (C - 1)
                vals = plsc.load_gather(stage, [rvec, cvec])
                plsc.store_scatter(buf, [cvec, jv], vals, mask=gv)
            return 0

        lax.fori_loop(0, (cnt + LANES - 1) >> 4, grp, 0)

    def cleanup(b_old, buf, cnt):
        def grp(g, _, b_old=b_old):
            gl = g * LANES + iota
            gv = gl < cnt
            e = ent[pl.ds(b_old * BS + g * LANES, LANES)]
            jv = jnp.where(gv, e & (BS - 1), 0)

            @plsc.parallel_loop(0, C, step=1, unroll=4)
            def _(k):
                cvec = (k + iota) & (C - 1)
                plsc.store_scatter(buf, [cvec, jv], zeros16, mask=gv)
            return 0

        lax.fori_loop(0, (cnt + LANES - 1) >> 4, grp, 0)

    gslots = ((idx0, stage0, buf0, gs0, os0), (idx1, stage1, buf1, gs1, os1))
    fill_idx(0, msave[0], idx0)
    g_desc = [None] * NB
    o_desc = [None] * NB
    g_desc[0] = pltpu.async_copy(feat_hbm.at[idx0], stage0, gs0)

    for b in range(NB):
        idxr, stage, buf, gs, osem = gslots[b % 2]
        if b + 1 < NB:
            idxn, stagen, _, gsn, _ = gslots[(b + 1) % 2]
            fill_idx(b + 1, msave[b + 1], idxn)
            g_desc[b + 1] = pltpu.async_copy(feat_hbm.at[idxn], stagen, gsn)
        if b >= 2:
            o_desc[b - 2].wait()
            cleanup(b - 2, buf, msave[b - 2])
        g_desc[b].wait()
        process(b, stage, buf, 0, jnp.minimum(msave[b], GC))

        nchunks = (msave[b] + GC - 1) // GC

        @pl.when(nchunks > 1)
        def _extra(b=b, idxr=idxr, stage=stage, buf=buf, gs=gs):
            def echunk(ci, _):
                base = ci * GC
                cntc = jnp.clip(msave[b] - base, 0, GC)

                def g(gi, _):
                    gl = gi * LANES + iota
                    e = ent[pl.ds(b * BS + base + gi * LANES, LANES)]
                    idxr[pl.ds(gi * LANES, LANES)] = \
                        jnp.where(gl < cntc, e >> 9, 0)
                    return 0

                lax.fori_loop(0, GC // LANES, g, 0)
                pltpu.async_copy(feat_hbm.at[idxr], stage, gs).wait()
                process(b, stage, buf, base, cntc)
                return 0

            lax.fori_loop(1, nchunks, echunk, 0)

        o_desc[b] = pltpu.async_copy(
            buf, out_hbm.at[:, pl.ds(r0 + b * BS, BS)], osem)

    o_desc[NB - 2].wait()
    o_desc[NB - 1].wait()


@jax.jit
def _scatter(xs, ys, feat):
    mesh = plsc.VectorSubcoreMesh(core_axis_name="c", subcore_axis_name="s",
                                  num_cores=NC, num_subcores=NS)
    return pl.kernel(
        _scatter_body,
        out_type=jax.ShapeDtypeStruct((C, HW), jnp.float32),
        mesh=mesh,
        compiler_params=pltpu.CompilerParams(needs_layout_passes=False,
                                             use_tc_tiling_on_sc=False),
        scratch_types=[
            pltpu.VMEM((CHUNK,), jnp.int32),
            pltpu.VMEM((CHUNK,), jnp.int32),
            pltpu.VMEM((CHUNK,), jnp.int32),
            pltpu.VMEM((CHUNK,), jnp.int32),
            pltpu.VMEM((TILE_RANGE,), jnp.int32),
            pltpu.VMEM((TILE_RANGE + LANES,), jnp.int32),
            pltpu.VMEM((GC,), jnp.int32),
            pltpu.VMEM((GC,), jnp.int32),
            pltpu.VMEM((GC, C), jnp.float32),
            pltpu.VMEM((GC, C), jnp.float32),
            pltpu.VMEM((C, BS), jnp.float32),
            pltpu.VMEM((C, BS), jnp.float32),
            pltpu.SemaphoreType.DMA,
            pltpu.SemaphoreType.DMA,
            pltpu.SemaphoreType.DMA,
            pltpu.SemaphoreType.DMA,
            pltpu.SemaphoreType.DMA,
            pltpu.SemaphoreType.DMA,
            pltpu.SemaphoreType.DMA,
            pltpu.SemaphoreType.DMA,
        ],
    )(xs, ys, feat)


def kernel(pillar_features, coords):
    coords = jnp.asarray(coords, jnp.int32)
    canvas = _scatter(coords[:, 0], coords[:, 1], pillar_features)
    return canvas.reshape(1, C, H, W)
